# baseline (device time: 134346 ns/iter reference)
import jax
import jax.numpy as jnp
from jax import lax
from jax.experimental import pallas as pl
from jax.experimental.pallas import tpu as pltpu

N_DEV = 8
SQ = 2048
SKV = 2048
D_MODEL = 1024
H_PER = 8
DH = 128
DQ = H_PER * DH
SCALE = 0.08838834764831843
CHUNK = SQ // N_DEV
QBLK = CHUNK


def kernel(x, Wq, K_ext, V_ext, Wo):
    my = lax.axis_index("i")
    xb = x.reshape(SQ, D_MODEL).astype(jnp.bfloat16)
    wq = lax.dynamic_slice(Wq, (0, my * DQ), (D_MODEL, DQ)).astype(jnp.bfloat16)
    wo = lax.dynamic_slice(Wo, (my * DQ, 0), (DQ, D_MODEL)).astype(jnp.bfloat16)
    k = K_ext.reshape(SKV, DQ).astype(jnp.bfloat16)
    v = V_ext.reshape(SKV, DQ).astype(jnp.bfloat16)

    def body(x_ref, wq_ref, k_ref, v_ref, wo_ref, out_ref,
             q_buf, ctx_buf, rs_buf, rs_send, rs_recv,
             agr_send, agr_recv, agl_send, agl_recv):
        me = lax.axis_index("i")
        right = lax.rem(me + 1, N_DEV)
        left = lax.rem(me + N_DEV - 1, N_DEV)

        def rs_step(s, carry):
            c = lax.rem(me + N_DEV - s, N_DEV)
            rows = pl.ds(c * QBLK, QBLK)
            q_buf[rows, :] = (
                jnp.dot(x_ref[rows, :], wq_ref[...],
                        preferred_element_type=jnp.float32) * SCALE
            ).astype(jnp.bfloat16)
            nqb = QBLK // 64
            qb = 4 * c + lax.broadcasted_iota(jnp.int32, (nqb, SKV), 0)
            kb = lax.broadcasted_iota(jnp.int32, (nqb, SKV), 1) // 64
            mask4 = (qb == kb) | (kb == 0) | (lax.rem(qb + kb, 3) == 0)
            bias4 = jnp.where(mask4, 0.0, -1e9).astype(jnp.float32)
            bias = jnp.broadcast_to(
                bias4[:, None, :], (nqb, 64, SKV)).reshape(QBLK, SKV)
            for h in range(H_PER):
                cols = slice(h * DH, (h + 1) * DH)
                sc = lax.dot_general(
                    q_buf[rows, cols], k_ref[:, cols],
                    (((1,), (1,)), ((), ())),
                    preferred_element_type=jnp.float32)
                w = jnp.exp(sc + bias).astype(jnp.bfloat16)
                ws = jnp.sum(w, axis=1, keepdims=True, dtype=jnp.float32)
                ctx = jnp.dot(w, v_ref[:, cols],
                              preferred_element_type=jnp.float32)
                ctx_buf[rows, cols] = (ctx / ws).astype(jnp.bfloat16)
            part = jnp.dot(ctx_buf[rows, :], wo_ref[...],
                           preferred_element_type=jnp.float32)

            sm1 = lax.max(s - 1, 0)
            cm1 = lax.rem(me + N_DEV - sm1, N_DEV)

            @pl.when(s > 0)
            def _():
                prev = pltpu.make_async_remote_copy(
                    src_ref=out_ref.at[pl.ds(cm1 * QBLK, QBLK), :],
                    dst_ref=rs_buf.at[sm1],
                    send_sem=rs_send.at[sm1],
                    recv_sem=rs_recv.at[sm1],
                    device_id=(right,),
                    device_id_type=pl.DeviceIdType.MESH,
                )
                prev.wait()

            inc = jnp.where(s > 0, rs_buf[sm1].astype(jnp.float32), 0.0)
            out_ref[rows, :] = (part + inc).astype(jnp.bfloat16)

            @pl.when(s < N_DEV - 1)
            def _():
                rdma = pltpu.make_async_remote_copy(
                    src_ref=out_ref.at[rows, :],
                    dst_ref=rs_buf.at[s],
                    send_sem=rs_send.at[s],
                    recv_sem=rs_recv.at[s],
                    device_id=(right,),
                    device_id_type=pl.DeviceIdType.MESH,
                )
                rdma.start()

            return carry

        lax.fori_loop(0, N_DEV, rs_step, 0)

        half = CHUNK // 2
        for t in range(4):
            sc_r = lax.rem(me + 1 + N_DEV - t, N_DEV)
            if t < 3:
                rows_r = pl.ds(sc_r * CHUNK, CHUNK)
            else:
                rows_r = pl.ds(sc_r * CHUNK, half)
            rd_r = pltpu.make_async_remote_copy(
                src_ref=out_ref.at[rows_r, :],
                dst_ref=out_ref.at[rows_r, :],
                send_sem=agr_send.at[t],
                recv_sem=agr_recv.at[t],
                device_id=(right,),
                device_id_type=pl.DeviceIdType.MESH,
            )
            rd_r.start()
            sc_l = lax.rem(me + 1 + t, N_DEV)
            if t < 3:
                rows_l = pl.ds(sc_l * CHUNK, CHUNK)
            else:
                rows_l = pl.ds(sc_l * CHUNK + half, half)
            rd_l = pltpu.make_async_remote_copy(
                src_ref=out_ref.at[rows_l, :],
                dst_ref=out_ref.at[rows_l, :],
                send_sem=agl_send.at[t],
                recv_sem=agl_recv.at[t],
                device_id=(left,),
                device_id_type=pl.DeviceIdType.MESH,
            )
            rd_l.start()
            rd_r.wait()
            rd_l.wait()

    out = pl.pallas_call(
        body,
        out_shape=jax.ShapeDtypeStruct((SQ, D_MODEL), jnp.bfloat16),
        in_specs=[pl.BlockSpec(memory_space=pltpu.VMEM)] * 5,
        out_specs=pl.BlockSpec(memory_space=pltpu.VMEM),
        scratch_shapes=[
            pltpu.VMEM((SQ, DQ), jnp.bfloat16),
            pltpu.VMEM((SQ, DQ), jnp.bfloat16),
            pltpu.VMEM((N_DEV - 1, CHUNK, D_MODEL), jnp.bfloat16),
            pltpu.SemaphoreType.DMA((N_DEV - 1,)),
            pltpu.SemaphoreType.DMA((N_DEV - 1,)),
            pltpu.SemaphoreType.DMA((4,)),
            pltpu.SemaphoreType.DMA((4,)),
            pltpu.SemaphoreType.DMA((4,)),
            pltpu.SemaphoreType.DMA((4,)),
        ],
        compiler_params=pltpu.CompilerParams(
            vmem_limit_bytes=100 * 1024 * 1024,
        ),
    )(xb, wq, k, v, wo)
    return out.reshape(1, SQ, D_MODEL)


# device time: 128409 ns/iter; 1.0462x vs baseline; 1.0462x over previous
import jax
import jax.numpy as jnp
from jax import lax
from jax.experimental import pallas as pl
from jax.experimental.pallas import tpu as pltpu

N_DEV = 8
SQ = 2048
SKV = 2048
D_MODEL = 1024
H_PER = 8
DH = 128
DQ = H_PER * DH
SCALE = 0.08838834764831843
CHUNK = SQ // N_DEV
QBLK = CHUNK


def kernel(x, Wq, K_ext, V_ext, Wo):
    my = lax.axis_index("i")
    xb = x.reshape(SQ, D_MODEL).astype(jnp.bfloat16)
    wq = lax.dynamic_slice(Wq, (0, my * DQ), (D_MODEL, DQ)).astype(jnp.bfloat16)
    wo = lax.dynamic_slice(Wo, (my * DQ, 0), (DQ, D_MODEL)).astype(jnp.bfloat16)
    k = K_ext.reshape(SKV, DQ).astype(jnp.bfloat16)
    v = V_ext.reshape(SKV, DQ).astype(jnp.bfloat16)

    def body(x_ref, wq_ref, k_ref, v_ref, wo_ref, out_ref,
             q_buf, ctx_buf, rs_buf, rs_send, rs_recv,
             agr_send, agr_recv, agl_send, agl_recv):
        me = lax.axis_index("i")
        right = lax.rem(me + 1, N_DEV)
        left = lax.rem(me + N_DEV - 1, N_DEV)

        def rs_step(s, carry):
            c = lax.rem(me + N_DEV - s, N_DEV)
            rows = pl.ds(c * QBLK, QBLK)
            q_buf[rows, :] = (
                jnp.dot(x_ref[rows, :], wq_ref[...],
                        preferred_element_type=jnp.float32) * SCALE
            ).astype(jnp.bfloat16)
            nqb = QBLK // 64
            qb = 4 * c + lax.broadcasted_iota(jnp.int32, (nqb, SKV), 0)
            kb = lax.broadcasted_iota(jnp.int32, (nqb, SKV), 1) // 64
            mask4 = (qb == kb) | (kb == 0) | (lax.rem(qb + kb, 3) == 0)
            bias4 = jnp.where(mask4, 0.0, -1e9).astype(jnp.float32)
            bias = jnp.broadcast_to(
                bias4[:, None, :], (nqb, 64, SKV)).reshape(QBLK, SKV)
            for h in range(H_PER):
                cols = slice(h * DH, (h + 1) * DH)
                sc = lax.dot_general(
                    q_buf[rows, cols], k_ref[:, cols],
                    (((1,), (1,)), ((), ())),
                    preferred_element_type=jnp.float32)
                w = jnp.exp(sc + bias)
                ws = jnp.sum(w, axis=1, keepdims=True)
                ctx = jnp.dot(w.astype(jnp.bfloat16), v_ref[:, cols],
                              preferred_element_type=jnp.float32)
                ctx_buf[rows, cols] = (ctx / ws).astype(jnp.bfloat16)
            out_ref[rows, :] = jnp.dot(
                ctx_buf[rows, :], wo_ref[...],
                preferred_element_type=jnp.float32).astype(jnp.bfloat16)

            sm1 = lax.max(s - 1, 0)
            cm1 = lax.rem(me + N_DEV - sm1, N_DEV)

            @pl.when(s > 0)
            def _():
                prev = pltpu.make_async_remote_copy(
                    src_ref=out_ref.at[pl.ds(cm1 * QBLK, QBLK), :],
                    dst_ref=rs_buf.at[sm1],
                    send_sem=rs_send.at[sm1],
                    recv_sem=rs_recv.at[sm1],
                    device_id=(right,),
                    device_id_type=pl.DeviceIdType.MESH,
                )
                prev.wait()
                out_ref[rows, :] = out_ref[rows, :] + rs_buf[sm1]

            @pl.when(s < N_DEV - 1)
            def _():
                rdma = pltpu.make_async_remote_copy(
                    src_ref=out_ref.at[rows, :],
                    dst_ref=rs_buf.at[s],
                    send_sem=rs_send.at[s],
                    recv_sem=rs_recv.at[s],
                    device_id=(right,),
                    device_id_type=pl.DeviceIdType.MESH,
                )
                rdma.start()

            return carry

        lax.fori_loop(0, N_DEV, rs_step, 0)

        half = CHUNK // 2
        for t in range(4):
            sc_r = lax.rem(me + 1 + N_DEV - t, N_DEV)
            if t < 3:
                rows_r = pl.ds(sc_r * CHUNK, CHUNK)
            else:
                rows_r = pl.ds(sc_r * CHUNK, half)
            rd_r = pltpu.make_async_remote_copy(
                src_ref=out_ref.at[rows_r, :],
                dst_ref=out_ref.at[rows_r, :],
                send_sem=agr_send.at[t],
                recv_sem=agr_recv.at[t],
                device_id=(right,),
                device_id_type=pl.DeviceIdType.MESH,
            )
            rd_r.start()
            sc_l = lax.rem(me + 1 + t, N_DEV)
            if t < 3:
                rows_l = pl.ds(sc_l * CHUNK, CHUNK)
            else:
                rows_l = pl.ds(sc_l * CHUNK + half, half)
            rd_l = pltpu.make_async_remote_copy(
                src_ref=out_ref.at[rows_l, :],
                dst_ref=out_ref.at[rows_l, :],
                send_sem=agl_send.at[t],
                recv_sem=agl_recv.at[t],
                device_id=(left,),
                device_id_type=pl.DeviceIdType.MESH,
            )
            rd_l.start()
            rd_r.wait()
            rd_l.wait()

    out = pl.pallas_call(
        body,
        out_shape=jax.ShapeDtypeStruct((SQ, D_MODEL), jnp.bfloat16),
        in_specs=[pl.BlockSpec(memory_space=pltpu.VMEM)] * 5,
        out_specs=pl.BlockSpec(memory_space=pltpu.VMEM),
        scratch_shapes=[
            pltpu.VMEM((SQ, DQ), jnp.bfloat16),
            pltpu.VMEM((SQ, DQ), jnp.bfloat16),
            pltpu.VMEM((N_DEV - 1, CHUNK, D_MODEL), jnp.bfloat16),
            pltpu.SemaphoreType.DMA((N_DEV - 1,)),
            pltpu.SemaphoreType.DMA((N_DEV - 1,)),
            pltpu.SemaphoreType.DMA((4,)),
            pltpu.SemaphoreType.DMA((4,)),
            pltpu.SemaphoreType.DMA((4,)),
            pltpu.SemaphoreType.DMA((4,)),
        ],
        compiler_params=pltpu.CompilerParams(
            vmem_limit_bytes=100 * 1024 * 1024,
        ),
    )(xb, wq, k, v, wo)
    return out.reshape(1, SQ, D_MODEL)


# device time: 124095 ns/iter; 1.0826x vs baseline; 1.0348x over previous
import jax
import jax.numpy as jnp
from jax import lax
from jax.experimental import pallas as pl
from jax.experimental.pallas import tpu as pltpu

N_DEV = 8
SQ = 2048
SKV = 2048
D_MODEL = 1024
H_PER = 8
DH = 128
DQ = H_PER * DH
SCALE = 0.08838834764831843
CHUNK = SQ // N_DEV
QBLK = CHUNK


def kernel(x, Wq, K_ext, V_ext, Wo):
    my = lax.axis_index("i")
    xb = x.reshape(SQ, D_MODEL).astype(jnp.bfloat16)
    wq = lax.dynamic_slice(Wq, (0, my * DQ), (D_MODEL, DQ)).astype(jnp.bfloat16)
    wo = lax.dynamic_slice(Wo, (my * DQ, 0), (DQ, D_MODEL)).astype(jnp.bfloat16)
    k = K_ext.reshape(SKV, DQ).astype(jnp.bfloat16)
    v = V_ext.reshape(SKV, DQ).astype(jnp.bfloat16)

    def body(x_ref, wq_ref, k_ref, v_ref, wo_ref, out_ref,
             q_buf, ctx_buf, rs_buf, rs_send, rs_recv,
             agr_send, agr_recv, agl_send, agl_recv):
        me = lax.axis_index("i")
        right = lax.rem(me + 1, N_DEV)
        left = lax.rem(me + N_DEV - 1, N_DEV)

        def rs_step(s, carry):
            c = lax.rem(me + N_DEV - s, N_DEV)
            rows = pl.ds(c * QBLK, QBLK)
            q_buf[rows, :] = (
                jnp.dot(x_ref[rows, :], wq_ref[...],
                        preferred_element_type=jnp.float32) * SCALE
            ).astype(jnp.bfloat16)
            nqb = QBLK // 64
            qb = 4 * c + lax.broadcasted_iota(jnp.int32, (nqb, SKV), 0)
            kb = lax.broadcasted_iota(jnp.int32, (nqb, SKV), 1) // 64
            mask4 = (qb == kb) | (kb == 0) | (lax.rem(qb + kb, 3) == 0)
            bias4 = jnp.where(mask4, 0.0, -1e9).astype(jnp.float32)
            bias = jnp.broadcast_to(
                bias4[:, None, :], (nqb, 64, SKV)).reshape(QBLK, SKV)
            for h in range(H_PER):
                cols = slice(h * DH, (h + 1) * DH)
                sc = lax.dot_general(
                    q_buf[rows, cols], k_ref[:, cols],
                    (((1,), (1,)), ((), ())),
                    preferred_element_type=jnp.float32)
                w = jnp.exp(sc + bias)
                ws = jnp.sum(w, axis=1, keepdims=True)
                ctx = jnp.dot(w.astype(jnp.bfloat16), v_ref[:, cols],
                              preferred_element_type=jnp.float32)
                ctx_buf[rows, cols] = (ctx / ws).astype(jnp.bfloat16)
            out_ref[rows, :] = jnp.dot(
                ctx_buf[rows, :], wo_ref[...],
                preferred_element_type=jnp.float32).astype(jnp.bfloat16)

            sm1 = lax.max(s - 1, 0)
            cm1 = lax.rem(me + N_DEV - sm1, N_DEV)

            @pl.when(s > 0)
            def _():
                prev = pltpu.make_async_remote_copy(
                    src_ref=out_ref.at[pl.ds(cm1 * QBLK, QBLK), :],
                    dst_ref=rs_buf.at[sm1],
                    send_sem=rs_send.at[sm1],
                    recv_sem=rs_recv.at[sm1],
                    device_id=(right,),
                    device_id_type=pl.DeviceIdType.MESH,
                )
                prev.wait()
                out_ref[rows, :] = out_ref[rows, :] + rs_buf[sm1]

            @pl.when(s < N_DEV - 1)
            def _():
                rdma = pltpu.make_async_remote_copy(
                    src_ref=out_ref.at[rows, :],
                    dst_ref=rs_buf.at[s],
                    send_sem=rs_send.at[s],
                    recv_sem=rs_recv.at[s],
                    device_id=(right,),
                    device_id_type=pl.DeviceIdType.MESH,
                )
                rdma.start()

            return carry

        lax.fori_loop(0, N_DEV, rs_step, 0)

        zpart = lax.rem(me + 4, N_DEV)
        own_rows = pl.ds(lax.rem(me + 1, N_DEV) * CHUNK, CHUNK)
        rd_z = pltpu.make_async_remote_copy(
            src_ref=out_ref.at[own_rows, :],
            dst_ref=out_ref.at[own_rows, :],
            send_sem=agr_send.at[3],
            recv_sem=agr_recv.at[3],
            device_id=(zpart,),
            device_id_type=pl.DeviceIdType.MESH,
        )
        rd_z.start()
        for t in range(3):
            sc_r = lax.rem(me + 1 + N_DEV - t, N_DEV)
            rows_r = pl.ds(sc_r * CHUNK, CHUNK)
            rd_r = pltpu.make_async_remote_copy(
                src_ref=out_ref.at[rows_r, :],
                dst_ref=out_ref.at[rows_r, :],
                send_sem=agr_send.at[t],
                recv_sem=agr_recv.at[t],
                device_id=(right,),
                device_id_type=pl.DeviceIdType.MESH,
            )
            rd_r.start()
            sc_l = lax.rem(me + 1 + t, N_DEV)
            rows_l = pl.ds(sc_l * CHUNK, CHUNK)
            rd_l = pltpu.make_async_remote_copy(
                src_ref=out_ref.at[rows_l, :],
                dst_ref=out_ref.at[rows_l, :],
                send_sem=agl_send.at[t],
                recv_sem=agl_recv.at[t],
                device_id=(left,),
                device_id_type=pl.DeviceIdType.MESH,
            )
            rd_l.start()
            rd_r.wait()
            rd_l.wait()
        rd_z.wait()

    out = pl.pallas_call(
        body,
        out_shape=jax.ShapeDtypeStruct((SQ, D_MODEL), jnp.bfloat16),
        in_specs=[pl.BlockSpec(memory_space=pltpu.VMEM)] * 5,
        out_specs=pl.BlockSpec(memory_space=pltpu.VMEM),
        scratch_shapes=[
            pltpu.VMEM((SQ, DQ), jnp.bfloat16),
            pltpu.VMEM((SQ, DQ), jnp.bfloat16),
            pltpu.VMEM((N_DEV - 1, CHUNK, D_MODEL), jnp.bfloat16),
            pltpu.SemaphoreType.DMA((N_DEV - 1,)),
            pltpu.SemaphoreType.DMA((N_DEV - 1,)),
            pltpu.SemaphoreType.DMA((4,)),
            pltpu.SemaphoreType.DMA((4,)),
            pltpu.SemaphoreType.DMA((4,)),
            pltpu.SemaphoreType.DMA((4,)),
        ],
        compiler_params=pltpu.CompilerParams(
            vmem_limit_bytes=100 * 1024 * 1024,
        ),
    )(xb, wq, k, v, wo)
    return out.reshape(1, SQ, D_MODEL)
